# baseline (speedup 1.0000x reference)
import functools

import jax
import jax.numpy as jnp
from jax import lax
from jax.experimental import pallas as pl
from jax.experimental.pallas import tpu as pltpu
from jax.experimental.pallas import tpu_sc as plsc

_NC = 2
_NS = 16
_NW = _NC * _NS
_L = 16

_CN = 16


def _pool_kernel(n, nodes, d):
    n_chunks = nodes // _CN
    tail_nodes = nodes - n_chunks * _CN
    chunks_per_w = n_chunks // _NW
    assert chunks_per_w * _NW == n_chunks and chunks_per_w % 2 == 0
    cw = 7 * _CN
    widx_words = chunks_per_w * cw
    d_vec = d // _L
    mesh = plsc.VectorSubcoreMesh(core_axis_name="c", subcore_axis_name="s")

    @functools.partial(
        pl.kernel,
        mesh=mesh,
        compiler_params=pltpu.CompilerParams(
            needs_layout_passes=False, use_tc_tiling_on_sc=False
        ),
        out_type=jax.ShapeDtypeStruct((nodes, d), jnp.float32),
        scratch_types=[
            pltpu.VMEM((_CN, d), jnp.float32),
            pltpu.VMEM((_CN, d), jnp.float32),
            pltpu.VMEM((7 * tail_nodes if tail_nodes else 1, d),
                       jnp.float32),
            pltpu.VMEM((widx_words,), jnp.int32),
            pltpu.VMEM((cw,), jnp.int32),
            pltpu.VMEM((cw,), jnp.int32),
            pltpu.SemaphoreType.DMA,
            pltpu.SemaphoreType.DMA,
            pltpu.SemaphoreType.DMA,
            pltpu.SemaphoreType.DMA,
        ],
    )
    def body(x_hbm, neigh_hbm, out_hbm, buf0, buf1, tbuf, rawidx,
             istage0, istage1, gsem0, gsem1, osem0, osem1):
        wid = lax.axis_index("s") * _NC + lax.axis_index("c")
        inv7 = jnp.float32(1.0 / 7.0)
        zero = jnp.zeros((_L,), jnp.float32)
        lane7 = lax.iota(jnp.int32, _L) * 7
        bufs = (buf0, buf1)
        istages = (istage0, istage1)
        gsems = (gsem0, gsem1)
        osems = (osem0, osem1)
        base = wid * chunks_per_w

        pltpu.sync_copy(neigh_hbm.at[pl.ds(wid * widx_words, widx_words)],
                        rawidx)

        def zero_buf(b):
            def per_c(ci, carry):
                off = ci * _L
                for j in range(_CN):
                    bufs[b][j, pl.ds(off, _L)] = zero
                return carry

            lax.fori_loop(0, d_vec, per_c, 0, unroll=False)

        def start_gathers(g, b):
            for k in range(7):
                ivec = plsc.load_gather(rawidx, [lane7 + (g * cw + k)])
                istages[b][pl.ds(k * _CN, _CN)] = ivec
            for k in range(7):
                pltpu.async_copy(
                    x_hbm.at[istages[b].at[pl.ds(k * _CN, _CN)]],
                    bufs[b], gsems[b], add=True)

        def drain_gathers(g, b):
            for k in range(7):
                pltpu.make_async_copy(
                    x_hbm.at[istages[b].at[pl.ds(k * _CN, _CN)]],
                    bufs[b], gsems[b]).wait()

        def scale_buf(b):
            def per_c(ci, carry):
                off = ci * _L
                for j in range(_CN):
                    bufs[b][j, pl.ds(off, _L)] = (
                        bufs[b][j, pl.ds(off, _L)] * inv7)
                return carry

            lax.fori_loop(0, d_vec, per_c, 0, unroll=False)

        def out_copy(g, b):
            return pltpu.make_async_copy(
                bufs[b], out_hbm.at[pl.ds((base + g) * _CN, _CN)], osems[b])

        zero_buf(0)
        start_gathers(0, 0)

        def per_iter(i, carry):
            for b in range(2):
                g = i * 2 + b
                nb = 1 - b
                @pl.when(g + 1 < chunks_per_w)
                def _():
                    @pl.when(g >= 1)
                    def _():
                        out_copy(g, nb).wait()
                    zero_buf(nb)
                    start_gathers(g + 1, nb)
                drain_gathers(g, b)
                scale_buf(b)
                out_copy(g, b).start()
            return carry

        lax.fori_loop(0, chunks_per_w // 2, per_iter, 0, unroll=False)

        for b in range(2):
            out_copy(0, b).wait()

        if tail_nodes:
            tail_words = 7 * tail_nodes

            @pl.when(wid == 0)
            def _():
                pltpu.sync_copy(
                    neigh_hbm.at[pl.ds(n_chunks * cw, tail_words)],
                    rawidx.at[pl.ds(0, tail_words)],
                )
                pltpu.async_copy(
                    x_hbm.at[rawidx.at[pl.ds(0, tail_words)]],
                    tbuf.at[pl.ds(0, tail_words)], gsem0,
                ).wait()

                def per_c(ci, carry):
                    off = ci * _L
                    for j in range(tail_nodes):
                        acc = tbuf[7 * j, pl.ds(off, _L)]
                        for k in range(1, 7):
                            acc = acc + tbuf[7 * j + k, pl.ds(off, _L)]
                        tbuf[j, pl.ds(off, _L)] = acc * inv7
                    return carry

                lax.fori_loop(0, d_vec, per_c, 0, unroll=False)
                pltpu.sync_copy(tbuf.at[pl.ds(0, tail_nodes)],
                                out_hbm.at[pl.ds(n_chunks * _CN, tail_nodes)])

    return body


def kernel(x, neigh_orders):
    B, C, N = x.shape
    nodes = (N + 6) // 4
    d = B * C

    xt = jnp.transpose(x, (2, 0, 1)).reshape(N, d)
    out = _pool_kernel(N, nodes, d)(xt, neigh_orders)
    return jnp.transpose(out.reshape(nodes, B, C), (1, 2, 0))


# speedup vs baseline: 1.0185x; 1.0185x over previous
import functools

import jax
import jax.numpy as jnp
from jax import lax
from jax.experimental import pallas as pl
from jax.experimental.pallas import tpu as pltpu
from jax.experimental.pallas import tpu_sc as plsc

_NC = 2
_NS = 16
_NW = _NC * _NS
_L = 16

_CN = 8


def _pool_kernel(n, nodes, d):
    n_chunks = nodes // _CN
    tail_nodes = nodes - n_chunks * _CN
    chunks_per_w = n_chunks // _NW
    assert chunks_per_w * _NW == n_chunks and chunks_per_w % 2 == 0
    cw = 7 * _CN
    widx_words = chunks_per_w * cw
    d_vec = d // _L
    mesh = plsc.VectorSubcoreMesh(core_axis_name="c", subcore_axis_name="s")

    @functools.partial(
        pl.kernel,
        mesh=mesh,
        compiler_params=pltpu.CompilerParams(
            needs_layout_passes=False, use_tc_tiling_on_sc=False
        ),
        out_type=jax.ShapeDtypeStruct((nodes, d), jnp.float32),
        scratch_types=[
            pltpu.VMEM((cw, d), jnp.float32),
            pltpu.VMEM((cw, d), jnp.float32),
            pltpu.VMEM((widx_words,), jnp.int32),
            pltpu.SemaphoreType.DMA,
            pltpu.SemaphoreType.DMA,
            pltpu.SemaphoreType.DMA,
            pltpu.SemaphoreType.DMA,
        ],
    )
    def body(x_hbm, neigh_hbm, out_hbm, buf0, buf1, rawidx,
             gsem0, gsem1, osem0, osem1):
        wid = lax.axis_index("s") * _NC + lax.axis_index("c")
        inv7 = jnp.float32(1.0 / 7.0)
        bufs = (buf0, buf1)
        gsems = (gsem0, gsem1)
        osems = (osem0, osem1)
        base = wid * chunks_per_w

        pltpu.sync_copy(neigh_hbm.at[pl.ds(wid * widx_words, widx_words)],
                        rawidx)

        def gather_parts(g, b):
            yield (x_hbm.at[rawidx.at[pl.ds(g * cw, 32)]],
                   bufs[b].at[pl.ds(0, 32)])
            yield (x_hbm.at[rawidx.at[pl.ds(g * cw + 32, 24)]],
                   bufs[b].at[pl.ds(32, 24)])

        def start_gather(g, b):
            for src, dst in gather_parts(g, b):
                pltpu.make_async_copy(src, dst, gsems[b]).start()

        def wait_gather(g, b):
            for src, dst in gather_parts(g, b):
                pltpu.make_async_copy(src, dst, gsems[b]).wait()

        def reduce_rows(buf, n_out):
            def per_c(ci, carry):
                off = ci * _L
                for j in range(n_out):
                    acc = buf[7 * j, pl.ds(off, _L)]
                    for k in range(1, 7):
                        acc = acc + buf[7 * j + k, pl.ds(off, _L)]
                    buf[j, pl.ds(off, _L)] = acc * inv7
                return carry

            lax.fori_loop(0, d_vec, per_c, 0, unroll=False)

        start_gather(0, 0)

        def per_iter(i, carry):
            for b in range(2):
                g = i * 2 + b
                nb = 1 - b
                @pl.when(g + 1 < chunks_per_w)
                def _():
                    @pl.when(g >= 1)
                    def _():
                        pltpu.make_async_copy(
                            bufs[nb].at[pl.ds(0, _CN)],
                            out_hbm.at[pl.ds((base + g) * _CN, _CN)],
                            osems[nb],
                        ).wait()
                    start_gather(g + 1, nb)
                wait_gather(g, b)
                reduce_rows(bufs[b], _CN)
                pltpu.make_async_copy(
                    bufs[b].at[pl.ds(0, _CN)],
                    out_hbm.at[pl.ds((base + g) * _CN, _CN)],
                    osems[b],
                ).start()
            return carry

        lax.fori_loop(0, chunks_per_w // 2, per_iter, 0, unroll=False)

        for b in range(2):
            pltpu.make_async_copy(
                bufs[b].at[pl.ds(0, _CN)],
                out_hbm.at[pl.ds(base * _CN, _CN)],
                osems[b],
            ).wait()

        if tail_nodes:
            tail_words = 7 * tail_nodes

            @pl.when(wid == 0)
            def _():
                pltpu.sync_copy(
                    neigh_hbm.at[pl.ds(n_chunks * cw, tail_words)],
                    rawidx.at[pl.ds(0, tail_words)],
                )
                pltpu.async_copy(
                    x_hbm.at[rawidx.at[pl.ds(0, tail_words)]],
                    buf0.at[pl.ds(0, tail_words)], gsem0,
                ).wait()
                reduce_rows(buf0, tail_nodes)
                pltpu.sync_copy(buf0.at[pl.ds(0, tail_nodes)],
                                out_hbm.at[pl.ds(n_chunks * _CN, tail_nodes)])

    return body


def kernel(x, neigh_orders):
    B, C, N = x.shape
    nodes = (N + 6) // 4
    d = B * C

    xt = jnp.transpose(x, (2, 0, 1)).reshape(N, d)
    out = _pool_kernel(N, nodes, d)(xt, neigh_orders)
    return jnp.transpose(out.reshape(nodes, B, C), (1, 2, 0))
